# baseline (speedup 1.0000x reference)
import jax
import jax.numpy as jnp
from jax.experimental import pallas as pl
from jax.experimental.pallas import tpu as pltpu

K = 32
N_FEATURES = 16384
D_IN = 1024
BATCH = 1024

BT = 128
FB = 1024
N_BT = BATCH // BT
N_FB = N_FEATURES // FB
DEPTH = 8
NEG = float("-inf")


def _encode_body(x_ref, b_ref, enc_ref, latent_ref, wa_ref, pre_scr, xc_scr,
                 stk_scr, t_scr, cmax_scr):
    b = pl.program_id(0)
    f = pl.program_id(1)

    @pl.when(f == 0)
    def _():
        xc_scr[...] = x_ref[...] - b_ref[...]

    @pl.when(f < N_FB)
    def _():
        blk = jnp.dot(xc_scr[...], enc_ref[...],
                      preferred_element_type=jnp.float32)
        pre_scr[:, pl.ds(pl.multiple_of(f * FB, FB), FB)] = blk

        @pl.when(f == 0)
        def _():
            stk_scr[0] = blk
            for l in range(1, DEPTH):
                stk_scr[l] = jnp.full((BT, FB), NEG, jnp.float32)

        @pl.when(f != 0)
        def _():
            v = blk
            for l in range(DEPTH):
                cur = stk_scr[l]
                stk_scr[l] = jnp.maximum(cur, v)
                v = jnp.minimum(cur, v)

    @pl.when(f == N_FB)
    def _():
        lane = jax.lax.broadcasted_iota(jnp.int32, (BT, FB), 1)
        a = stk_scr[0]
        idx_parts = []
        for _ in range(K):
            j = jnp.argmax(a, axis=1).astype(jnp.int32)[:, None]
            idx_parts.append(j)
            a = jnp.where(lane == j, NEG, a)
        idx = jnp.concatenate(idx_parts, axis=1)
        cands_parts = []
        for l in range(DEPTH):
            acc = jnp.full((BT, K), NEG, jnp.float32)
            for bi in range(FB // 128):
                sub = stk_scr[l][:, bi * 128:(bi + 1) * 128]
                li = jnp.clip(idx - bi * 128, 0, 127)
                g = jnp.take_along_axis(sub, li, axis=1,
                                        mode="promise_in_bounds")
                valid = (idx >= bi * 128) & (idx < (bi + 1) * 128)
                acc = jnp.maximum(acc, jnp.where(valid, g, NEG))
            cands_parts.append(acc)
        cands = jnp.concatenate(cands_parts, axis=1)
        t = jnp.max(cands, axis=1, keepdims=True)
        for _ in range(K - 1):
            t = jnp.max(jnp.where(cands < t, cands, NEG), axis=1,
                        keepdims=True)
        t_scr[...] = jnp.broadcast_to(t, (BT, 128))

    @pl.when(f > N_FB)
    def _():
        c = f - (N_FB + 1)
        col = pl.ds(pl.multiple_of(c * FB, FB), FB)
        chunk = pre_scr[:, col]
        t = t_scr[:, :1]
        latw = jnp.where(chunk >= t, chunk, 0.0)
        latent_ref[:, col] = latw
        cm = jnp.max(latw, axis=0, keepdims=True)

        @pl.when(b == 0)
        def _():
            cmax_scr[:, col] = cm

        @pl.when(b != 0)
        def _():
            cmax_scr[:, col] = jnp.maximum(cmax_scr[:, col], cm)

    @pl.when(jnp.logical_and(b == N_BT - 1, f == 2 * N_FB))
    def _():
        wa_ref[...] = (cmax_scr[...] > 0.001).astype(jnp.int32)


def _decode_body(latent_ref, dec_ref, b_ref, rec_ref, acc_scr):
    k = pl.program_id(1)

    blk = jnp.dot(latent_ref[...], dec_ref[...],
                  preferred_element_type=jnp.float32)

    @pl.when(k == 0)
    def _():
        acc_scr[...] = blk

    @pl.when(k != 0)
    def _():
        acc_scr[...] += blk

    @pl.when(k == N_FB - 1)
    def _():
        rec_ref[...] = acc_scr[...] + b_ref[...]


@jax.jit
def kernel(x, pre_b, enc, dec):
    b2d = pre_b.reshape(1, D_IN)

    latent, wa = pl.pallas_call(
        _encode_body,
        grid=(N_BT, 2 * N_FB + 1),
        in_specs=[
            pl.BlockSpec((BT, D_IN), lambda b, f: (b, 0)),
            pl.BlockSpec((1, D_IN), lambda b, f: (0, 0)),
            pl.BlockSpec((D_IN, FB), lambda b, f: (0, jnp.minimum(f, N_FB - 1))),
        ],
        out_specs=[
            pl.BlockSpec((BT, N_FEATURES), lambda b, f: (b, 0)),
            pl.BlockSpec((1, N_FEATURES), lambda b, f: (0, 0)),
        ],
        out_shape=[
            jax.ShapeDtypeStruct((BATCH, N_FEATURES), jnp.float32),
            jax.ShapeDtypeStruct((1, N_FEATURES), jnp.int32),
        ],
        scratch_shapes=[
            pltpu.VMEM((BT, N_FEATURES), jnp.float32),
            pltpu.VMEM((BT, D_IN), jnp.float32),
            pltpu.VMEM((DEPTH, BT, FB), jnp.float32),
            pltpu.VMEM((BT, 128), jnp.float32),
            pltpu.VMEM((1, N_FEATURES), jnp.float32),
        ],
        compiler_params=pltpu.CompilerParams(
            dimension_semantics=("parallel", "arbitrary"),
        ),
    )(x, b2d, enc)

    rec = pl.pallas_call(
        _decode_body,
        grid=(N_BT, N_FB),
        in_specs=[
            pl.BlockSpec((BT, FB), lambda b, k: (b, k)),
            pl.BlockSpec((FB, D_IN), lambda b, k: (k, 0)),
            pl.BlockSpec((1, D_IN), lambda b, k: (0, 0)),
        ],
        out_specs=pl.BlockSpec((BT, D_IN), lambda b, k: (b, 0)),
        out_shape=jax.ShapeDtypeStruct((BATCH, D_IN), jnp.float32),
        scratch_shapes=[
            pltpu.VMEM((BT, D_IN), jnp.float32),
        ],
        compiler_params=pltpu.CompilerParams(
            dimension_semantics=("arbitrary", "arbitrary"),
        ),
    )(latent.astype(jnp.bfloat16), dec.astype(jnp.bfloat16), b2d)

    was_active = wa[0].astype(bool)
    return rec, latent, was_active


# speedup vs baseline: 1.0271x; 1.0271x over previous
import jax
import jax.numpy as jnp
from jax.experimental import pallas as pl
from jax.experimental.pallas import tpu as pltpu

K = 32
N_FEATURES = 16384
D_IN = 1024
BATCH = 1024

BT = 128
FB = 1024
N_BT = BATCH // BT
N_FB = N_FEATURES // FB
DEPTH = 8
NEG = float("-inf")


def _encode_body(x_ref, b_ref, enc_ref, latent_ref, wa_ref, pre_scr, xc_scr,
                 stk_scr, t_scr, cmax_scr):
    b = pl.program_id(0)
    f = pl.program_id(1)

    @pl.when(f == 0)
    def _():
        xc_scr[...] = x_ref[...] - b_ref[...]

    @pl.when(f < N_FB)
    def _():
        blk = jnp.dot(xc_scr[...], enc_ref[...],
                      preferred_element_type=jnp.float32)
        pre_scr[:, pl.ds(pl.multiple_of(f * FB, FB), FB)] = blk

        @pl.when(f == 0)
        def _():
            stk_scr[0] = blk
            for l in range(1, DEPTH):
                stk_scr[l] = jnp.full((BT, FB), NEG, jnp.float32)

        @pl.when(f != 0)
        def _():
            v = blk
            for l in range(DEPTH):
                cur = stk_scr[l]
                stk_scr[l] = jnp.maximum(cur, v)
                v = jnp.minimum(cur, v)

    @pl.when(f == N_FB)
    def _():
        lane = jax.lax.broadcasted_iota(jnp.int32, (BT, FB), 1)
        a = stk_scr[0]
        idx_parts = []
        for _ in range(K):
            j = jnp.argmax(a, axis=1).astype(jnp.int32)[:, None]
            idx_parts.append(j)
            a = jnp.where(lane == j, NEG, a)
        idx = jnp.concatenate(idx_parts, axis=1)
        cands_parts = []
        for l in range(DEPTH):
            acc = jnp.full((BT, K), NEG, jnp.float32)
            for bi in range(FB // 128):
                sub = stk_scr[l][:, bi * 128:(bi + 1) * 128]
                li = jnp.clip(idx - bi * 128, 0, 127)
                g = jnp.take_along_axis(sub, li, axis=1,
                                        mode="promise_in_bounds")
                valid = (idx >= bi * 128) & (idx < (bi + 1) * 128)
                acc = jnp.maximum(acc, jnp.where(valid, g, NEG))
            cands_parts.append(acc)
        cands = jnp.concatenate(cands_parts, axis=1)
        t = jnp.max(cands, axis=1, keepdims=True)
        for _ in range(K - 1):
            t = jnp.max(jnp.where(cands < t, cands, NEG), axis=1,
                        keepdims=True)
        t_scr[...] = jnp.broadcast_to(t, (BT, 128))

    @pl.when(f > N_FB)
    def _():
        c = f - (N_FB + 1)
        col = pl.ds(pl.multiple_of(c * FB, FB), FB)
        chunk = pre_scr[:, col]
        t = t_scr[:, :1]
        latw = jnp.where(chunk >= t, chunk, 0.0)
        latent_ref[:, col] = latw
        cm = jnp.max(latw, axis=0, keepdims=True)

        @pl.when(b == 0)
        def _():
            cmax_scr[:, col] = cm

        @pl.when(b != 0)
        def _():
            cmax_scr[:, col] = jnp.maximum(cmax_scr[:, col], cm)

    @pl.when(jnp.logical_and(b == N_BT - 1, f == 2 * N_FB))
    def _():
        wa_ref[...] = (cmax_scr[...] > 0.001).astype(jnp.int32)


def _decode_body(latent_ref, dec_ref, b_ref, rec_ref, acc_scr):
    k = pl.program_id(1)

    blk = jnp.dot(latent_ref[...], dec_ref[...],
                  preferred_element_type=jnp.float32)

    @pl.when(k == 0)
    def _():
        acc_scr[...] = blk

    @pl.when(k != 0)
    def _():
        acc_scr[...] += blk

    @pl.when(k == N_FB - 1)
    def _():
        rec_ref[...] = acc_scr[...] + b_ref[...]


@jax.jit
def kernel(x, pre_b, enc, dec):
    b2d = pre_b.reshape(1, D_IN)

    latent, wa = pl.pallas_call(
        _encode_body,
        grid=(N_BT, 2 * N_FB + 1),
        in_specs=[
            pl.BlockSpec((BT, D_IN), lambda b, f: (b, 0)),
            pl.BlockSpec((1, D_IN), lambda b, f: (0, 0)),
            pl.BlockSpec((D_IN, FB), lambda b, f: (0, jnp.minimum(f, N_FB - 1))),
        ],
        out_specs=[
            pl.BlockSpec((BT, N_FEATURES), lambda b, f: (b, 0)),
            pl.BlockSpec((1, N_FEATURES), lambda b, f: (0, 0)),
        ],
        out_shape=[
            jax.ShapeDtypeStruct((BATCH, N_FEATURES), jnp.float32),
            jax.ShapeDtypeStruct((1, N_FEATURES), jnp.int32),
        ],
        scratch_shapes=[
            pltpu.VMEM((BT, N_FEATURES), jnp.float32),
            pltpu.VMEM((BT, D_IN), jnp.float32),
            pltpu.VMEM((DEPTH, BT, FB), jnp.float32),
            pltpu.VMEM((BT, 128), jnp.float32),
            pltpu.VMEM((1, N_FEATURES), jnp.float32),
        ],
        compiler_params=pltpu.CompilerParams(
            dimension_semantics=("parallel", "arbitrary"),
        ),
    )(x, b2d, enc)

    rec = pl.pallas_call(
        _decode_body,
        grid=(N_BT, N_FB),
        in_specs=[
            pl.BlockSpec((BT, FB), lambda b, k: (b, k)),
            pl.BlockSpec((FB, D_IN), lambda b, k: (k, 0)),
            pl.BlockSpec((1, D_IN), lambda b, k: (0, 0)),
        ],
        out_specs=pl.BlockSpec((BT, D_IN), lambda b, k: (b, 0)),
        out_shape=jax.ShapeDtypeStruct((BATCH, D_IN), jnp.float32),
        scratch_shapes=[
            pltpu.VMEM((BT, D_IN), jnp.float32),
        ],
        compiler_params=pltpu.CompilerParams(
            dimension_semantics=("arbitrary", "arbitrary"),
        ),
    )(latent, dec, b2d)

    was_active = wa[0].astype(bool)
    return rec, latent, was_active


# speedup vs baseline: 1.5708x; 1.5294x over previous
import jax
import jax.numpy as jnp
from jax.experimental import pallas as pl
from jax.experimental.pallas import tpu as pltpu

K = 32
N_FEATURES = 16384
D_IN = 1024
BATCH = 1024

BT = 256
FB = 1024
N_BT = BATCH // BT
N_FB = N_FEATURES // FB
DEPTH = 8
NEG = float("-inf")


def _encode_body(x_ref, b_ref, enc_ref, latent_ref, wa_ref, pre_scr, xc_scr,
                 stk_scr, t_scr, cmax_scr):
    b = pl.program_id(0)
    f = pl.program_id(1)

    @pl.when(f == 0)
    def _():
        xc_scr[...] = x_ref[...] - b_ref[...]

    @pl.when(f < N_FB)
    def _():
        blk = jnp.dot(xc_scr[...], enc_ref[...],
                      preferred_element_type=jnp.float32)
        pre_scr[:, pl.ds(pl.multiple_of(f * FB, FB), FB)] = blk

        @pl.when(f == 0)
        def _():
            stk_scr[0] = blk
            for l in range(1, DEPTH):
                stk_scr[l] = jnp.full((BT, FB), NEG, jnp.float32)

        @pl.when(f != 0)
        def _():
            v = blk
            for l in range(DEPTH):
                cur = stk_scr[l]
                stk_scr[l] = jnp.maximum(cur, v)
                v = jnp.minimum(cur, v)

    @pl.when(f == N_FB)
    def _():
        lane = jax.lax.broadcasted_iota(jnp.int32, (BT, FB), 1)
        a = stk_scr[0]
        idx_parts = []
        for _ in range(K):
            j = jnp.argmax(a, axis=1).astype(jnp.int32)[:, None]
            idx_parts.append(j)
            a = jnp.where(lane == j, NEG, a)
        idx = jnp.concatenate(idx_parts, axis=1)
        cands_parts = []
        for l in range(DEPTH):
            acc = jnp.full((BT, K), NEG, jnp.float32)
            for bi in range(FB // 128):
                sub = stk_scr[l][:, bi * 128:(bi + 1) * 128]
                li = jnp.clip(idx - bi * 128, 0, 127)
                g = jnp.take_along_axis(sub, li, axis=1,
                                        mode="promise_in_bounds")
                valid = (idx >= bi * 128) & (idx < (bi + 1) * 128)
                acc = jnp.maximum(acc, jnp.where(valid, g, NEG))
            cands_parts.append(acc)
        cands = jnp.concatenate(cands_parts, axis=1)
        t = jnp.max(cands, axis=1, keepdims=True)
        for _ in range(K - 1):
            t = jnp.max(jnp.where(cands < t, cands, NEG), axis=1,
                        keepdims=True)
        t_scr[...] = jnp.broadcast_to(t, (BT, 128))

    @pl.when(f > N_FB)
    def _():
        c = f - (N_FB + 1)
        col = pl.ds(pl.multiple_of(c * FB, FB), FB)
        chunk = pre_scr[:, col]
        t = t_scr[:, :1]
        latw = jnp.where(chunk >= t, chunk, 0.0)
        latent_ref[...] = latw
        cm = jnp.max(latw, axis=0, keepdims=True)

        @pl.when(b == 0)
        def _():
            cmax_scr[:, col] = cm

        @pl.when(b != 0)
        def _():
            cmax_scr[:, col] = jnp.maximum(cmax_scr[:, col], cm)

    @pl.when(jnp.logical_and(b == N_BT - 1, f == 2 * N_FB))
    def _():
        wa_ref[...] = (cmax_scr[...] > 0.001).astype(jnp.int32)


def _decode_body(latent_ref, dec_ref, b_ref, rec_ref, acc_scr):
    k = pl.program_id(1)

    blk = jnp.dot(latent_ref[...], dec_ref[...],
                  preferred_element_type=jnp.float32)

    @pl.when(k == 0)
    def _():
        acc_scr[...] = blk

    @pl.when(k != 0)
    def _():
        acc_scr[...] += blk

    @pl.when(k == N_FB - 1)
    def _():
        rec_ref[...] = acc_scr[...] + b_ref[...]


@jax.jit
def kernel(x, pre_b, enc, dec):
    b2d = pre_b.reshape(1, D_IN)

    latent, wa = pl.pallas_call(
        _encode_body,
        grid=(N_BT, 2 * N_FB + 1),
        in_specs=[
            pl.BlockSpec((BT, D_IN), lambda b, f: (b, 0)),
            pl.BlockSpec((1, D_IN), lambda b, f: (0, 0)),
            pl.BlockSpec((D_IN, FB), lambda b, f: (0, jnp.minimum(f, N_FB - 1))),
        ],
        out_specs=[
            pl.BlockSpec((BT, FB),
                         lambda b, f: (b, jnp.clip(f - (N_FB + 1), 0,
                                                   N_FB - 1))),
            pl.BlockSpec((1, N_FEATURES), lambda b, f: (0, 0)),
        ],
        out_shape=[
            jax.ShapeDtypeStruct((BATCH, N_FEATURES), jnp.float32),
            jax.ShapeDtypeStruct((1, N_FEATURES), jnp.int32),
        ],
        scratch_shapes=[
            pltpu.VMEM((BT, N_FEATURES), jnp.float32),
            pltpu.VMEM((BT, D_IN), jnp.float32),
            pltpu.VMEM((DEPTH, BT, FB), jnp.float32),
            pltpu.VMEM((BT, 128), jnp.float32),
            pltpu.VMEM((1, N_FEATURES), jnp.float32),
        ],
        compiler_params=pltpu.CompilerParams(
            dimension_semantics=("parallel", "arbitrary"),
        ),
    )(x, b2d, enc)

    rec = pl.pallas_call(
        _decode_body,
        grid=(N_BT, N_FB),
        in_specs=[
            pl.BlockSpec((BT, FB), lambda b, k: (b, k)),
            pl.BlockSpec((FB, D_IN), lambda b, k: (k, 0)),
            pl.BlockSpec((1, D_IN), lambda b, k: (0, 0)),
        ],
        out_specs=pl.BlockSpec((BT, D_IN), lambda b, k: (b, 0)),
        out_shape=jax.ShapeDtypeStruct((BATCH, D_IN), jnp.float32),
        scratch_shapes=[
            pltpu.VMEM((BT, D_IN), jnp.float32),
        ],
        compiler_params=pltpu.CompilerParams(
            dimension_semantics=("arbitrary", "arbitrary"),
        ),
    )(latent, dec, b2d)

    was_active = wa[0].astype(bool)
    return rec, latent, was_active
